# Initial kernel scaffold; baseline (speedup 1.0000x reference)
import functools

import jax
import jax.numpy as jnp
from jax import lax
from jax.experimental import pallas as pl
from jax.experimental.pallas import tpu as pltpu
from jax.experimental.pallas import tpu_sc as plsc

N = 10000
D = 128
NNZ = 320000
NP = 10112
NC, NS = 2, 16
NW = NC * NS
CB = 128
KW = 80
NNZ_P = NW * KW * CB
NROW = NNZ_P // CB
KH = NNZ_P // NS // CB
RPS = NP // NS
R = NP // 8

_mesh = plsc.VectorSubcoreMesh(core_axis_name="c", subcore_axis_name="s")


def _zero_fill(buf, nrows, ncols):
    @pl.loop(0, nrows)
    def _(r):
        @pl.loop(0, ncols // 16)
        def _(q):
            buf[r, pl.ds(q * 16, 16)] = jnp.zeros((16,), jnp.float32)


def _sc_propagate_body(src_sel, table_hbm, eidx_hbm, out_hbm,
                       src_v, dst_v, bufa, bufb, acc, sema, semb):
    c = lax.axis_index("c")
    s = lax.axis_index("s")
    w = c * NS + s
    _zero_fill(bufa, CB, D)
    base = s * RPS
    @pl.loop(0, 4)
    def _(k):
        pltpu.sync_copy(bufa, acc.at[pl.ds(base + k * CB, CB)])
    pltpu.sync_copy(bufa.at[pl.ds(0, RPS - 4 * CB)],
                    acc.at[pl.ds(base + 4 * CB, RPS - 4 * CB)])
    pltpu.sync_copy(eidx_hbm.at[src_sel, pl.ds(w * KW, KW)], src_v)
    pltpu.sync_copy(eidx_hbm.at[1 - src_sel, pl.ds(w * KW, KW)], dst_v)
    plsc.subcore_barrier()

    def start(j, buf, sem):
        pltpu.async_copy(table_hbm.at[src_v.at[j]], buf, sem)

    def wait(j, buf, sem):
        pltpu.make_async_copy(table_hbm.at[src_v.at[j]], buf, sem).wait()

    def scat(j, buf):
        pltpu.sync_copy(buf, acc.at[dst_v.at[j]], add=True)

    start(0, bufa, sema)
    @pl.loop(0, KW // 2)
    def _(p):
        j0 = p * 2
        wait(j0, bufa, sema)
        start(j0 + 1, bufb, semb)
        scat(j0, bufa)
        wait(j0 + 1, bufb, semb)
        @pl.when(j0 + 2 < KW)
        def _():
            start(j0 + 2, bufa, sema)
        scat(j0 + 1, bufb)

    plsc.subcore_barrier()
    @pl.loop(0, 4)
    def _(k):
        pltpu.sync_copy(acc.at[pl.ds(base + k * CB, CB)],
                        out_hbm.at[c, pl.ds(base + k * CB, CB)])
    pltpu.sync_copy(acc.at[pl.ds(base + 4 * CB, RPS - 4 * CB)],
                    out_hbm.at[c, pl.ds(base + 4 * CB, RPS - 4 * CB)])


def _make_propagate(src_sel):
    return pl.kernel(
        functools.partial(_sc_propagate_body, src_sel),
        out_type=jax.ShapeDtypeStruct((NC, NP, D), jnp.float32),
        mesh=_mesh,
        scratch_types=[
            pltpu.VMEM((KW, CB), jnp.int32),
            pltpu.VMEM((KW, CB), jnp.int32),
            pltpu.VMEM((CB, D), jnp.float32),
            pltpu.VMEM((CB, D), jnp.float32),
            pltpu.VMEM_SHARED((NP, D), jnp.float32),
            pltpu.SemaphoreType.DMA,
            pltpu.SemaphoreType.DMA,
        ],
    )


_sc_n2e = _make_propagate(0)
_sc_e2n = _make_propagate(1)


@functools.partial(
    pl.kernel,
    out_type=jax.ShapeDtypeStruct((NC, NP, 16), jnp.float32),
    mesh=_mesh,
    scratch_types=[
        pltpu.VMEM((KH, CB), jnp.int32),
        pltpu.VMEM((CB, 16), jnp.float32),
        pltpu.VMEM((RPS, 16), jnp.float32),
        pltpu.VMEM_SHARED((NP, 16), jnp.float32),
    ],
)
def _sc_degrees(eidx_hbm, out_hbm, idx_v, ones_v, zbuf, acc):
    c = lax.axis_index("c")
    s = lax.axis_index("s")
    @pl.loop(0, CB)
    def _(r):
        ones_v[r, pl.ds(0, 16)] = jnp.ones((16,), jnp.float32)
    _zero_fill(zbuf, RPS, 16)
    base = s * RPS
    pltpu.sync_copy(zbuf, acc.at[pl.ds(base, RPS)])
    pltpu.sync_copy(eidx_hbm.at[c, pl.ds(s * KH, KH)], idx_v)
    plsc.subcore_barrier()
    @pl.loop(0, KH)
    def _(j):
        pltpu.sync_copy(ones_v, acc.at[idx_v.at[j]], add=True)
    plsc.subcore_barrier()
    pltpu.sync_copy(acc.at[pl.ds(base, RPS)], out_hbm.at[c, pl.ds(base, RPS)])


def _dinv_sqrt(hist_ref, idx):
    deg = hist_ref[idx, :, 0:1]
    return jnp.where(deg > 0, 1.0 / jnp.sqrt(deg), 0.0)


def _tc_xw_scale_body(x_ref, w_ref, hist_ref, o_ref):
    ds_ = _dinv_sqrt(hist_ref, 0)
    o_ref[...] = jnp.dot(x_ref[...] * ds_, w_ref[...],
                         preferred_element_type=jnp.float32,
                         precision=lax.Precision.HIGHEST)


def _tc_combine_binv_body(p_ref, hist_ref, o_ref):
    e = hist_ref[0, :, 0:1]
    binv = jnp.where(e > 0, 1.0 / e, 0.0)
    o_ref[...] = (p_ref[0] + p_ref[1]) * binv


def _tc_layer2_head_body(p_ref, hist_ref, b1_ref, w2_ref, o_ref):
    ds_ = _dinv_sqrt(hist_ref, 0)
    h = jnp.maximum((p_ref[0] + p_ref[1]) * ds_ + b1_ref[...], 0.0)
    o_ref[...] = jnp.dot(h * ds_, w2_ref[...],
                         preferred_element_type=jnp.float32,
                         precision=lax.Precision.HIGHEST)


def _tc_final_body(p_ref, hist_ref, b2_ref, o_ref):
    ds_ = _dinv_sqrt(hist_ref, 0)
    o_ref[...] = (p_ref[0] + p_ref[1]) * ds_ + b2_ref[...]


_row_spec = pl.BlockSpec((R, D), lambda i: (i, 0))
_w_spec = pl.BlockSpec((D, D), lambda i: (0, 0))
_b_spec = pl.BlockSpec((1, D), lambda i: (0, 0))
_part_spec = pl.BlockSpec((2, R, D), lambda i: (0, i, 0))


def _hist_spec(which):
    return pl.BlockSpec((1, R, 16), lambda i, _w=which: (_w, i, 0))


_OUT = jax.ShapeDtypeStruct((NP, D), jnp.float32)
_G = NP // R

_tc_xw_scale = pl.pallas_call(
    _tc_xw_scale_body, grid=(_G,), out_shape=_OUT,
    in_specs=[_row_spec, _w_spec, _hist_spec(0)], out_specs=_row_spec)
_tc_combine_binv = pl.pallas_call(
    _tc_combine_binv_body, grid=(_G,), out_shape=_OUT,
    in_specs=[_part_spec, _hist_spec(1)], out_specs=_row_spec)
_tc_layer2_head = pl.pallas_call(
    _tc_layer2_head_body, grid=(_G,), out_shape=_OUT,
    in_specs=[_part_spec, _hist_spec(0), _b_spec, _w_spec],
    out_specs=_row_spec)
_tc_final = pl.pallas_call(
    _tc_final_body, grid=(_G,), out_shape=_OUT,
    in_specs=[_part_spec, _hist_spec(0), _b_spec], out_specs=_row_spec)


def kernel(x, edge_index, W1, b1, W2, b2):
    npad = NNZ_P - NNZ
    pad_ids = (N + jnp.arange(npad, dtype=jnp.int32) % (NP - N))
    eidx = jnp.concatenate(
        [edge_index, jnp.broadcast_to(pad_ids, (2, npad))], axis=1
    ).reshape(2, NROW, CB)
    xp = jnp.concatenate([x, jnp.zeros((NP - N, D), x.dtype)])
    b1r = b1.reshape(1, D)
    b2r = b2.reshape(1, D)

    hist = _sc_degrees(eidx)
    xn1 = _tc_xw_scale(xp, W1, hist)
    e1 = _sc_n2e(xn1, eidx)
    ef1 = _tc_combine_binv(e1, hist)
    n1 = _sc_e2n(ef1, eidx)
    xn2 = _tc_layer2_head(n1, hist, b1r, W2)
    e2 = _sc_n2e(xn2, eidx)
    ef2 = _tc_combine_binv(e2, hist)
    n2 = _sc_e2n(ef2, eidx)
    out = _tc_final(n2, hist, b2r)
    return out[:N]


# speedup vs baseline: 5.1378x; 5.1378x over previous
import functools

import jax
import jax.numpy as jnp
from jax import lax
from jax.experimental import pallas as pl
from jax.experimental.pallas import tpu as pltpu
from jax.experimental.pallas import tpu_sc as plsc

N = 10000
D = 128
NNZ = 320000
NP = 10112
NC, NS = 2, 16
NW = NC * NS
CB = 128
KW = 80
IB = 16
NNZ_P = NW * KW * CB
NROW = NNZ_P // CB
KH = NNZ_P // NS // CB
RPS = NP // NS
R = NP // 8

_mesh = plsc.VectorSubcoreMesh(core_axis_name="c", subcore_axis_name="s")


def _zero_fill(buf, nrows, ncols):
    @pl.loop(0, nrows)
    def _(r):
        @pl.loop(0, ncols // 16)
        def _(q):
            buf[r, pl.ds(q * 16, 16)] = jnp.zeros((16,), jnp.float32)


def _sc_propagate_body(src_sel, table_hbm, eidx_hbm, out_hbm,
                       src_v, dst_v, bufa, bufb, acc, sema, semb):
    c = lax.axis_index("c")
    s = lax.axis_index("s")
    w = c * NS + s
    _zero_fill(bufa, CB, D)
    base = s * RPS
    @pl.loop(0, 4)
    def _(k):
        pltpu.sync_copy(bufa, acc.at[pl.ds(base + k * CB, CB)])
    pltpu.sync_copy(bufa.at[pl.ds(0, RPS - 4 * CB)],
                    acc.at[pl.ds(base + 4 * CB, RPS - 4 * CB)])
    plsc.subcore_barrier()

    @pl.loop(0, KW)
    def _(j):
        r = w * KW + j
        pltpu.sync_copy(eidx_hbm.at[src_sel, r], src_v)
        pltpu.sync_copy(eidx_hbm.at[1 - src_sel, r], dst_v)
        pltpu.async_copy(table_hbm.at[src_v], bufa, sema).wait()
        pltpu.sync_copy(bufa, acc.at[dst_v], add=True)

    plsc.subcore_barrier()
    @pl.loop(0, 4)
    def _(k):
        pltpu.sync_copy(acc.at[pl.ds(base + k * CB, CB)],
                        out_hbm.at[c, pl.ds(base + k * CB, CB)])
    pltpu.sync_copy(acc.at[pl.ds(base + 4 * CB, RPS - 4 * CB)],
                    out_hbm.at[c, pl.ds(base + 4 * CB, RPS - 4 * CB)])


def _make_propagate(src_sel):
    return pl.kernel(
        functools.partial(_sc_propagate_body, src_sel),
        out_type=jax.ShapeDtypeStruct((NC, NP, D), jnp.float32),
        mesh=_mesh,
        scratch_types=[
            pltpu.VMEM((CB,), jnp.int32),
            pltpu.VMEM((CB,), jnp.int32),
            pltpu.VMEM((CB, D), jnp.float32),
            pltpu.VMEM((CB, D), jnp.float32),
            pltpu.VMEM_SHARED((NP, D), jnp.float32),
            pltpu.SemaphoreType.DMA,
            pltpu.SemaphoreType.DMA,
        ],
    )


_sc_n2e = _make_propagate(0)
_sc_e2n = _make_propagate(1)


@functools.partial(
    pl.kernel,
    out_type=jax.ShapeDtypeStruct((NC, NP, D), jnp.float32),
    mesh=_mesh,
    scratch_types=[
        pltpu.VMEM((CB,), jnp.int32),
        pltpu.VMEM((CB, D), jnp.float32),
        pltpu.VMEM_SHARED((NP, D), jnp.float32),
    ],
)
def _sc_degrees(eidx_hbm, out_hbm, idx_v, ones_v, acc):
    c = lax.axis_index("c")
    s = lax.axis_index("s")
    _zero_fill(ones_v, CB, D)
    base = s * RPS
    @pl.loop(0, 4)
    def _(k):
        pltpu.sync_copy(ones_v, acc.at[pl.ds(base + k * CB, CB)])
    pltpu.sync_copy(ones_v.at[pl.ds(0, RPS - 4 * CB)],
                    acc.at[pl.ds(base + 4 * CB, RPS - 4 * CB)])
    @pl.loop(0, CB)
    def _(r):
        @pl.loop(0, D // 16)
        def _(q):
            ones_v[r, pl.ds(q * 16, 16)] = jnp.ones((16,), jnp.float32)
    plsc.subcore_barrier()
    @pl.loop(0, KH)
    def _(j):
        pltpu.sync_copy(eidx_hbm.at[c, s * KH + j], idx_v)
        pltpu.sync_copy(ones_v, acc.at[idx_v], add=True)
    plsc.subcore_barrier()
    pltpu.sync_copy(acc.at[pl.ds(base, RPS)], out_hbm.at[c, pl.ds(base, RPS)])


def _dinv_sqrt(hist_ref, idx):
    deg = hist_ref[idx, :, 0:1]
    return jnp.where(deg > 0, 1.0 / jnp.sqrt(deg), 0.0)


def _tc_xw_scale_body(x_ref, w_ref, hist_ref, o_ref):
    ds_ = _dinv_sqrt(hist_ref, 0)
    o_ref[...] = jnp.dot(x_ref[...] * ds_, w_ref[...],
                         preferred_element_type=jnp.float32,
                         precision=lax.Precision.HIGHEST)


def _tc_combine_binv_body(p_ref, hist_ref, o_ref):
    e = hist_ref[0, :, 0:1]
    binv = jnp.where(e > 0, 1.0 / e, 0.0)
    o_ref[...] = (p_ref[0] + p_ref[1]) * binv


def _tc_layer2_head_body(p_ref, hist_ref, b1_ref, w2_ref, o_ref):
    ds_ = _dinv_sqrt(hist_ref, 0)
    h = jnp.maximum((p_ref[0] + p_ref[1]) * ds_ + b1_ref[...], 0.0)
    o_ref[...] = jnp.dot(h * ds_, w2_ref[...],
                         preferred_element_type=jnp.float32,
                         precision=lax.Precision.HIGHEST)


def _tc_final_body(p_ref, hist_ref, b2_ref, o_ref):
    ds_ = _dinv_sqrt(hist_ref, 0)
    o_ref[...] = (p_ref[0] + p_ref[1]) * ds_ + b2_ref[...]


_row_spec = pl.BlockSpec((R, D), lambda i: (i, 0))
_w_spec = pl.BlockSpec((D, D), lambda i: (0, 0))
_b_spec = pl.BlockSpec((1, D), lambda i: (0, 0))
_part_spec = pl.BlockSpec((2, R, D), lambda i: (0, i, 0))


def _hist_spec(which):
    return pl.BlockSpec((1, R, D), lambda i, _w=which: (_w, i, 0))


_OUT = jax.ShapeDtypeStruct((NP, D), jnp.float32)
_G = NP // R

_tc_xw_scale = pl.pallas_call(
    _tc_xw_scale_body, grid=(_G,), out_shape=_OUT,
    in_specs=[_row_spec, _w_spec, _hist_spec(0)], out_specs=_row_spec)
_tc_combine_binv = pl.pallas_call(
    _tc_combine_binv_body, grid=(_G,), out_shape=_OUT,
    in_specs=[_part_spec, _hist_spec(1)], out_specs=_row_spec)
_tc_layer2_head = pl.pallas_call(
    _tc_layer2_head_body, grid=(_G,), out_shape=_OUT,
    in_specs=[_part_spec, _hist_spec(0), _b_spec, _w_spec],
    out_specs=_row_spec)
_tc_final = pl.pallas_call(
    _tc_final_body, grid=(_G,), out_shape=_OUT,
    in_specs=[_part_spec, _hist_spec(0), _b_spec], out_specs=_row_spec)


def kernel(x, edge_index, W1, b1, W2, b2):
    npad = NNZ_P - NNZ
    pad_ids = (N + jnp.arange(npad, dtype=jnp.int32) % (NP - N))
    eidx = jnp.concatenate(
        [edge_index, jnp.broadcast_to(pad_ids, (2, npad))], axis=1
    ).reshape(2, NROW, CB)
    xp = jnp.concatenate([x, jnp.zeros((NP - N, D), x.dtype)])
    b1r = b1.reshape(1, D)
    b2r = b2.reshape(1, D)

    hist = _sc_degrees(eidx)
    xn1 = _tc_xw_scale(xp, W1, hist)
    e1 = _sc_n2e(xn1, eidx)
    ef1 = _tc_combine_binv(e1, hist)
    n1 = _sc_e2n(ef1, eidx)
    xn2 = _tc_layer2_head(n1, hist, b1r, W2)
    e2 = _sc_n2e(xn2, eidx)
    ef2 = _tc_combine_binv(e2, hist)
    n2 = _sc_e2n(ef2, eidx)
    out = _tc_final(n2, hist, b2r)
    return out[:N]


# speedup vs baseline: 8.1238x; 1.5812x over previous
import functools

import jax
import jax.numpy as jnp
from jax import lax
from jax.experimental import pallas as pl
from jax.experimental.pallas import tpu as pltpu
from jax.experimental.pallas import tpu_sc as plsc

N = 10000
D = 128
NNZ = 320000
NP = 10112
NC, NS = 2, 16
NW = NC * NS
CB = 128
KW = 80
IB = 16
NNZ_P = NW * KW * CB
NROW = NNZ_P // CB
KH = NNZ_P // NS // CB
RPS = NP // NS
R = NP // 8

_mesh = plsc.VectorSubcoreMesh(core_axis_name="c", subcore_axis_name="s")


def _zero_fill(buf, nrows, ncols):
    @pl.loop(0, nrows)
    def _(r):
        @pl.loop(0, ncols // 16)
        def _(q):
            buf[r, pl.ds(q * 16, 16)] = jnp.zeros((16,), jnp.float32)


def _sc_propagate_body(src_sel, table_hbm, eidx_hbm, out_hbm,
                       srca, dsta, srcb, dstb, bufa, bufb, acc, sema, semb):
    c = lax.axis_index("c")
    s = lax.axis_index("s")
    w = c * NS + s
    _zero_fill(bufa, CB, D)
    base = s * RPS
    @pl.loop(0, 4)
    def _(k):
        pltpu.sync_copy(bufa, acc.at[pl.ds(base + k * CB, CB)])
    pltpu.sync_copy(bufa.at[pl.ds(0, RPS - 4 * CB)],
                    acc.at[pl.ds(base + 4 * CB, RPS - 4 * CB)])
    plsc.subcore_barrier()

    def load_idx(j, sv, dv):
        r = jnp.minimum(w * KW + j, NW * KW - 1)
        pltpu.sync_copy(eidx_hbm.at[src_sel, r], sv)
        pltpu.sync_copy(eidx_hbm.at[1 - src_sel, r], dv)

    def start(sv, buf, sem):
        pltpu.async_copy(table_hbm.at[sv], buf, sem)

    def wait(sv, buf, sem):
        pltpu.make_async_copy(table_hbm.at[sv], buf, sem).wait()

    def scat(dv, buf):
        pltpu.sync_copy(buf, acc.at[dv], add=True)

    load_idx(0, srca, dsta)
    start(srca, bufa, sema)
    load_idx(1, srcb, dstb)

    @pl.loop(0, KW // 2)
    def _(p):
        j0 = p * 2
        wait(srca, bufa, sema)
        start(srcb, bufb, semb)
        scat(dsta, bufa)
        load_idx(j0 + 2, srca, dsta)
        wait(srcb, bufb, semb)
        @pl.when(j0 + 2 < KW)
        def _():
            start(srca, bufa, sema)
        scat(dstb, bufb)
        @pl.when(j0 + 3 < KW)
        def _():
            load_idx(j0 + 3, srcb, dstb)

    plsc.subcore_barrier()
    @pl.loop(0, 4)
    def _(k):
        pltpu.sync_copy(acc.at[pl.ds(base + k * CB, CB)],
                        out_hbm.at[c, pl.ds(base + k * CB, CB)])
    pltpu.sync_copy(acc.at[pl.ds(base + 4 * CB, RPS - 4 * CB)],
                    out_hbm.at[c, pl.ds(base + 4 * CB, RPS - 4 * CB)])


def _make_propagate(src_sel):
    return pl.kernel(
        functools.partial(_sc_propagate_body, src_sel),
        out_type=jax.ShapeDtypeStruct((NC, NP, D), jnp.float32),
        mesh=_mesh,
        scratch_types=[
            pltpu.VMEM((CB,), jnp.int32),
            pltpu.VMEM((CB,), jnp.int32),
            pltpu.VMEM((CB,), jnp.int32),
            pltpu.VMEM((CB,), jnp.int32),
            pltpu.VMEM((CB, D), jnp.float32),
            pltpu.VMEM((CB, D), jnp.float32),
            pltpu.VMEM_SHARED((NP, D), jnp.float32),
            pltpu.SemaphoreType.DMA,
            pltpu.SemaphoreType.DMA,
        ],
    )


_sc_n2e = _make_propagate(0)
_sc_e2n = _make_propagate(1)


@functools.partial(
    pl.kernel,
    out_type=jax.ShapeDtypeStruct((NC, NP, D), jnp.float32),
    mesh=_mesh,
    scratch_types=[
        pltpu.VMEM((CB,), jnp.int32),
        pltpu.VMEM((CB,), jnp.int32),
        pltpu.VMEM((CB, D), jnp.float32),
        pltpu.VMEM_SHARED((NP, D), jnp.float32),
        pltpu.SemaphoreType.DMA,
    ],
)
def _sc_degrees(eidx_hbm, out_hbm, ia, ib, ones_v, acc, semi):
    c = lax.axis_index("c")
    s = lax.axis_index("s")
    _zero_fill(ones_v, CB, D)
    base = s * RPS
    @pl.loop(0, 4)
    def _(k):
        pltpu.sync_copy(ones_v, acc.at[pl.ds(base + k * CB, CB)])
    pltpu.sync_copy(ones_v.at[pl.ds(0, RPS - 4 * CB)],
                    acc.at[pl.ds(base + 4 * CB, RPS - 4 * CB)])
    @pl.loop(0, CB)
    def _(r):
        @pl.loop(0, D // 16)
        def _(q):
            ones_v[r, pl.ds(q * 16, 16)] = jnp.ones((16,), jnp.float32)
    plsc.subcore_barrier()

    def istart(j, iv):
        pltpu.async_copy(eidx_hbm.at[c, s * KH + j], iv, semi)

    def iwait(j, iv):
        pltpu.make_async_copy(eidx_hbm.at[c, s * KH + j], iv, semi).wait()

    istart(0, ia)
    iwait(0, ia)

    @pl.loop(0, KH // 2)
    def _(p):
        j0 = p * 2
        istart(j0 + 1, ib)
        pltpu.sync_copy(ones_v, acc.at[ia], add=True)
        iwait(j0 + 1, ib)
        @pl.when(j0 + 2 < KH)
        def _():
            istart(j0 + 2, ia)
        pltpu.sync_copy(ones_v, acc.at[ib], add=True)
        @pl.when(j0 + 2 < KH)
        def _():
            iwait(j0 + 2, ia)
    plsc.subcore_barrier()
    pltpu.sync_copy(acc.at[pl.ds(base, RPS)], out_hbm.at[c, pl.ds(base, RPS)])


def _dinv_sqrt(hist_ref, idx):
    deg = hist_ref[idx, :, 0:1]
    return jnp.where(deg > 0, 1.0 / jnp.sqrt(deg), 0.0)


def _tc_xw_scale_body(x_ref, w_ref, hist_ref, o_ref):
    ds_ = _dinv_sqrt(hist_ref, 0)
    o_ref[...] = jnp.dot(x_ref[...] * ds_, w_ref[...],
                         preferred_element_type=jnp.float32,
                         precision=lax.Precision.HIGHEST)


def _tc_combine_binv_body(p_ref, hist_ref, o_ref):
    e = hist_ref[0, :, 0:1]
    binv = jnp.where(e > 0, 1.0 / e, 0.0)
    o_ref[...] = (p_ref[0] + p_ref[1]) * binv


def _tc_layer2_head_body(p_ref, hist_ref, b1_ref, w2_ref, o_ref):
    ds_ = _dinv_sqrt(hist_ref, 0)
    h = jnp.maximum((p_ref[0] + p_ref[1]) * ds_ + b1_ref[...], 0.0)
    o_ref[...] = jnp.dot(h * ds_, w2_ref[...],
                         preferred_element_type=jnp.float32,
                         precision=lax.Precision.HIGHEST)


def _tc_final_body(p_ref, hist_ref, b2_ref, o_ref):
    ds_ = _dinv_sqrt(hist_ref, 0)
    o_ref[...] = (p_ref[0] + p_ref[1]) * ds_ + b2_ref[...]


_row_spec = pl.BlockSpec((R, D), lambda i: (i, 0))
_w_spec = pl.BlockSpec((D, D), lambda i: (0, 0))
_b_spec = pl.BlockSpec((1, D), lambda i: (0, 0))
_part_spec = pl.BlockSpec((2, R, D), lambda i: (0, i, 0))


def _hist_spec(which):
    return pl.BlockSpec((1, R, D), lambda i, _w=which: (_w, i, 0))


_OUT = jax.ShapeDtypeStruct((NP, D), jnp.float32)
_G = NP // R

_tc_xw_scale = pl.pallas_call(
    _tc_xw_scale_body, grid=(_G,), out_shape=_OUT,
    in_specs=[_row_spec, _w_spec, _hist_spec(0)], out_specs=_row_spec)
_tc_combine_binv = pl.pallas_call(
    _tc_combine_binv_body, grid=(_G,), out_shape=_OUT,
    in_specs=[_part_spec, _hist_spec(1)], out_specs=_row_spec)
_tc_layer2_head = pl.pallas_call(
    _tc_layer2_head_body, grid=(_G,), out_shape=_OUT,
    in_specs=[_part_spec, _hist_spec(0), _b_spec, _w_spec],
    out_specs=_row_spec)
_tc_final = pl.pallas_call(
    _tc_final_body, grid=(_G,), out_shape=_OUT,
    in_specs=[_part_spec, _hist_spec(0), _b_spec], out_specs=_row_spec)


def kernel(x, edge_index, W1, b1, W2, b2):
    npad = NNZ_P - NNZ
    pad_ids = (N + jnp.arange(npad, dtype=jnp.int32) % (NP - N))
    eidx = jnp.concatenate(
        [edge_index, jnp.broadcast_to(pad_ids, (2, npad))], axis=1
    ).reshape(2, NROW, CB)
    xp = jnp.concatenate([x, jnp.zeros((NP - N, D), x.dtype)])
    b1r = b1.reshape(1, D)
    b2r = b2.reshape(1, D)

    hist = _sc_degrees(eidx)
    xn1 = _tc_xw_scale(xp, W1, hist)
    e1 = _sc_n2e(xn1, eidx)
    ef1 = _tc_combine_binv(e1, hist)
    n1 = _sc_e2n(ef1, eidx)
    xn2 = _tc_layer2_head(n1, hist, b1r, W2)
    e2 = _sc_n2e(xn2, eidx)
    ef2 = _tc_combine_binv(e2, hist)
    n2 = _sc_e2n(ef2, eidx)
    out = _tc_final(n2, hist, b2r)
    return out[:N]
